# baseline (speedup 1.0000x reference)
import functools

import jax
import jax.numpy as jnp
from jax import lax
from jax.experimental import pallas as pl
from jax.experimental.pallas import tpu as pltpu
from jax.experimental.pallas import tpu_sc as plsc

_EPS = 1e-9
_L = 16
_SC_CORES = 2
_SC_SUBCORES = 8
_KB = 5
_W = 128 * _KB


def _make_sc_call(B: int, N: int, C: int, b0: int):
    info = plsc.get_sparse_core_info()
    NC, NS = _SC_CORES, _SC_SUBCORES
    NW = NC * NS
    FULL = N // 128
    assert FULL % _KB == 0
    SU = FULL // _KB
    nb = B - b0
    n_su = nb * SU
    trips = -(-n_su // NW)
    ttrips = -(-nb // NW)

    mesh = plsc.VectorSubcoreMesh(
        core_axis_name="c", subcore_axis_name="s",
        num_cores=_SC_CORES, num_subcores=_SC_SUBCORES)

    @functools.partial(
        pl.kernel,
        out_type=(
            jax.ShapeDtypeStruct((NW, _L), jnp.float32),
            jax.ShapeDtypeStruct((NW, _L), jnp.float32),
        ),
        mesh=mesh,
        compiler_params=pltpu.CompilerParams(
            needs_layout_passes=False, use_tc_tiling_on_sc=True),
        scratch_types=(
            pltpu.VMEM((C, _W), jnp.float32),
            pltpu.VMEM((C, _W), jnp.float32),
            pltpu.VMEM((4, _W), jnp.float32),
            pltpu.VMEM((4, _W), jnp.float32),
            pltpu.VMEM((4, _W), jnp.float32),
            pltpu.VMEM((4, _W), jnp.float32),
            pltpu.VMEM((_W,), jnp.float32),
            pltpu.VMEM((_W,), jnp.float32),
            pltpu.VMEM((_L,), jnp.float32),
            pltpu.VMEM((_L,), jnp.float32),
            pltpu.SemaphoreType.DMA,
            pltpu.SemaphoreType.DMA,
        ),
    )
    def sc_call(ts_hbm, pb_hbm, tb_hbm, mk_hbm,
                tst_hbm, pbt_hbm, tbt_hbm, mkt_hbm,
                loss_out, ts_out,
                xv0, xv1, pv0, pv1, tv0, tv1, mv0, mv1,
                l_v, t_v, sem0, sem1):
        wid = lax.axis_index("s") * NC + lax.axis_index("c")
        slots = ((xv0, pv0, tv0, mv0, sem0), (xv1, pv1, tv1, mv1, sem1))

        def unit_refs(u):
            u = jnp.minimum(u, n_su - 1)
            b = b0 + u // SU
            col = pl.multiple_of((u - (u // SU) * SU) * _W, 128)
            return (ts_hbm.at[b, :, pl.ds(col, _W)],
                    pb_hbm.at[b, :, pl.ds(col, _W)],
                    tb_hbm.at[b, :, pl.ds(col, _W)],
                    mk_hbm.at[b, pl.ds(col, _W)])

        def start(u, slot):
            xv, pv, tv, mv, sem = slots[slot]
            s_ts, s_pb, s_tb, s_mk = unit_refs(u)
            pltpu.async_copy(s_ts, xv, sem)
            pltpu.async_copy(s_pb, pv, sem)
            pltpu.async_copy(s_tb, tv, sem)
            pltpu.async_copy(s_mk, mv, sem)

        def wait(u, slot):
            xv, pv, tv, mv, sem = slots[slot]
            s_ts, s_pb, s_tb, s_mk = unit_refs(u)
            pltpu.make_async_copy(s_ts, xv, sem).wait()
            pltpu.make_async_copy(s_pb, pv, sem).wait()
            pltpu.make_async_copy(s_tb, tv, sem).wait()
            pltpu.make_async_copy(s_mk, mv, sem).wait()

        def block_sums(xv, base):
            UR = 8
            assert C % UR == 0

            def row_body(j, accs):
                r = j * UR
                out = []
                for g in range(8):
                    s = base + g * _L
                    x = [xv[r + i, pl.ds(s, _L)] for i in range(UR)]
                    t = ((x[0] + x[1]) + (x[2] + x[3])) + (
                        (x[4] + x[5]) + (x[6] + x[7]))
                    out.append(accs[g] + t)
                return tuple(out)

            zeros = tuple(jnp.zeros((_L,), jnp.float32) for _ in range(8))
            return lax.fori_loop(0, C // UR, row_body, zeros)

        def block_loss(slot_refs, base, ws, acc_l, acc_t):
            _, pv, tv, mv, _ = slot_refs
            for g in range(8):
                s = base + g * _L
                w = ws[g]
                px1 = pv[0, pl.ds(s, _L)]
                py1 = pv[1, pl.ds(s, _L)]
                px2 = pv[2, pl.ds(s, _L)]
                py2 = pv[3, pl.ds(s, _L)]
                tx1 = tv[0, pl.ds(s, _L)]
                ty1 = tv[1, pl.ds(s, _L)]
                tx2 = tv[2, pl.ds(s, _L)]
                ty2 = tv[3, pl.ds(s, _L)]
                iw = jnp.maximum(jnp.minimum(px2, tx2) - jnp.maximum(px1, tx1), 0.0)
                ih = jnp.maximum(jnp.minimum(py2, ty2) - jnp.maximum(py1, ty1), 0.0)
                inter = iw * ih
                area1 = jnp.maximum(px2 - px1, 0.0) * jnp.maximum(py2 - py1, 0.0)
                area2 = jnp.maximum(tx2 - tx1, 0.0) * jnp.maximum(ty2 - ty1, 0.0)
                union = area1 + area2 - inter + _EPS
                iou = inter / union
                cw = jnp.maximum(px2, tx2) - jnp.minimum(px1, tx1)
                ch = jnp.maximum(py2, ty2) - jnp.minimum(py1, ty1)
                c_area = cw * ch + _EPS
                giou = iou - (c_area - union) / c_area
                m = mv[pl.ds(s, _L)]
                acc_l = acc_l + (1.0 - giou) * w * m
                acc_t = acc_t + w
            return acc_l, acc_t

        def compute_slab(slot, valid, acc_l, acc_t):
            srefs = slots[slot]
            xv = srefs[0]

            def kk_body(kk, carry):
                acc_l, acc_t = carry
                base = kk * 128
                ws = block_sums(xv, base)
                ws = tuple(w * valid for w in ws)
                return block_loss(srefs, base, ws, acc_l, acc_t)

            return lax.fori_loop(0, _KB, kk_body, (acc_l, acc_t))

        def valid_of(u):
            return jnp.where(u < n_su, jnp.float32(1.0), jnp.float32(0.0))

        start(wid, 0)
        start(wid + NW, 1)

        def pair_body(j, carry):
            acc_l, acc_t = carry
            for slot in (0, 1):
                i = 2 * j + slot
                uu = wid + i * NW
                wait(uu, slot)

                @pl.when(i + 2 < trips)
                def _():
                    start(uu + 2 * NW, slot)

                acc_l, acc_t = compute_slab(slot, valid_of(uu), acc_l, acc_t)
            return acc_l, acc_t

        zero = jnp.zeros((_L,), jnp.float32)
        acc = lax.fori_loop(0, trips // 2, pair_body, (zero, zero))
        if trips % 2:
            acc_l, acc_t = acc
            wait(wid + (trips - 1) * NW, 0)
            acc = compute_slab(0, valid_of(wid + (trips - 1) * NW), acc_l, acc_t)

        def tail_body(i, carry):
            acc_l, acc_t = carry
            t = wid + i * NW
            b = jnp.minimum(t, nb - 1)
            xv, pv, tv, mv, _ = slots[0]
            pltpu.sync_copy(tst_hbm.at[b], xv.at[:, pl.ds(0, 128)])
            pltpu.sync_copy(pbt_hbm.at[b], pv.at[:, pl.ds(0, 128)])
            pltpu.sync_copy(tbt_hbm.at[b], tv.at[:, pl.ds(0, 128)])
            pltpu.sync_copy(mkt_hbm.at[b], mv.at[pl.ds(0, 128)])
            ws = block_sums(xv, 0)
            valid = jnp.where(t < nb, jnp.float32(1.0), jnp.float32(0.0))
            ws = tuple(w * valid for w in ws)
            return block_loss(slots[0], 0, ws, acc_l, acc_t)

        acc_l, acc_t = lax.fori_loop(0, ttrips, tail_body, acc)
        l_v[...] = acc_l
        t_v[...] = acc_t
        pltpu.sync_copy(l_v, loss_out.at[wid])
        pltpu.sync_copy(t_v, ts_out.at[wid])

    return sc_call


def _make_tc_call(B: int, N: int, C: int, btc: int):
    BB = 8

    def body(ts_ref, pb_ref, tb_ref, mk_ref, lo_ref, to_ref):
        b = pl.program_id(0)

        @pl.when(b == 0)
        def _():
            lo_ref[...] = jnp.zeros_like(lo_ref)
            to_ref[...] = jnp.zeros_like(to_ref)

        w = jnp.sum(ts_ref[...], axis=1)
        m = mk_ref[...]
        px1 = pb_ref[:, 0, :]
        py1 = pb_ref[:, 1, :]
        px2 = pb_ref[:, 2, :]
        py2 = pb_ref[:, 3, :]
        tx1 = tb_ref[:, 0, :]
        ty1 = tb_ref[:, 1, :]
        tx2 = tb_ref[:, 2, :]
        ty2 = tb_ref[:, 3, :]
        iw = jnp.maximum(jnp.minimum(px2, tx2) - jnp.maximum(px1, tx1), 0.0)
        ih = jnp.maximum(jnp.minimum(py2, ty2) - jnp.maximum(py1, ty1), 0.0)
        inter = iw * ih
        area1 = jnp.maximum(px2 - px1, 0.0) * jnp.maximum(py2 - py1, 0.0)
        area2 = jnp.maximum(tx2 - tx1, 0.0) * jnp.maximum(ty2 - ty1, 0.0)
        union = area1 + area2 - inter + _EPS
        iou = inter / union
        cw = jnp.maximum(px2, tx2) - jnp.minimum(px1, tx1)
        ch = jnp.maximum(py2, ty2) - jnp.minimum(py1, ty1)
        c_area = cw * ch + _EPS
        giou = iou - (c_area - union) / c_area
        lo_ref[...] += (1.0 - giou) * w * m
        to_ref[...] += w

    return pl.pallas_call(
        body,
        grid=(btc // BB,),
        compiler_params=pltpu.CompilerParams(
            vmem_limit_bytes=100 * 1024 * 1024),
        in_specs=[
            pl.BlockSpec((BB, C, N), lambda b: (b, 0, 0)),
            pl.BlockSpec((BB, 4, N), lambda b: (b, 0, 0)),
            pl.BlockSpec((BB, 4, N), lambda b: (b, 0, 0)),
            pl.BlockSpec((BB, N), lambda b: (b, 0)),
        ],
        out_specs=[
            pl.BlockSpec((BB, N), lambda b: (0, 0)),
            pl.BlockSpec((BB, N), lambda b: (0, 0)),
        ],
        out_shape=[
            jax.ShapeDtypeStruct((BB, N), jnp.float32),
            jax.ShapeDtypeStruct((BB, N), jnp.float32),
        ],
    )


_B_TC = 48


def kernel(pred_bboxes, target_bboxes, target_scores, mask_positive):
    B, N, C = target_scores.shape
    FULL = N // 128
    ncols = FULL * 128
    ts_t = target_scores.transpose(0, 2, 1)
    pb_t = pred_bboxes.transpose(0, 2, 1)
    tb_t = target_bboxes.transpose(0, 2, 1)
    mask_f = mask_positive.astype(jnp.float32)

    def pad_tail(x):
        tail = x[..., ncols:]
        return jnp.pad(tail, [(0, 0)] * (x.ndim - 1) + [(0, 128 - (N - ncols))])

    loss_p, ts_p = _make_sc_call(B, N, C, _B_TC)(
        ts_t, pb_t, tb_t, mask_f,
        pad_tail(ts_t[_B_TC:]), pad_tail(pb_t[_B_TC:]),
        pad_tail(tb_t[_B_TC:]), pad_tail(mask_f[_B_TC:]))
    lo_tc, to_tc = _make_tc_call(B, N, C, _B_TC)(
        ts_t, pb_t, tb_t, mask_f)
    loss = loss_p.sum() + lo_tc.sum()
    ts = ts_p.sum() + to_tc.sum()
    return jnp.where(ts > 1.0, loss / ts, loss)


# speedup vs baseline: 1.0269x; 1.0269x over previous
import functools

import jax
import jax.numpy as jnp
from jax import lax
from jax.experimental import pallas as pl
from jax.experimental.pallas import tpu as pltpu
from jax.experimental.pallas import tpu_sc as plsc

_EPS = 1e-9
_L = 16
_SC_CORES = 1
_SC_SUBCORES = 16
_KB = 5
_W = 128 * _KB


def _make_sc_call(B: int, N: int, C: int, b0: int):
    info = plsc.get_sparse_core_info()
    NC, NS = _SC_CORES, _SC_SUBCORES
    NW = NC * NS
    FULL = N // 128
    assert FULL % _KB == 0
    SU = FULL // _KB
    nb = B - b0
    n_su = nb * SU
    trips = -(-n_su // NW)
    ttrips = -(-nb // NW)

    mesh = plsc.VectorSubcoreMesh(
        core_axis_name="c", subcore_axis_name="s",
        num_cores=_SC_CORES, num_subcores=_SC_SUBCORES)

    @functools.partial(
        pl.kernel,
        out_type=(
            jax.ShapeDtypeStruct((NW, _L), jnp.float32),
            jax.ShapeDtypeStruct((NW, _L), jnp.float32),
        ),
        mesh=mesh,
        compiler_params=pltpu.CompilerParams(
            needs_layout_passes=False, use_tc_tiling_on_sc=True),
        scratch_types=(
            pltpu.VMEM((C, _W), jnp.float32),
            pltpu.VMEM((C, _W), jnp.float32),
            pltpu.VMEM((4, _W), jnp.float32),
            pltpu.VMEM((4, _W), jnp.float32),
            pltpu.VMEM((4, _W), jnp.float32),
            pltpu.VMEM((4, _W), jnp.float32),
            pltpu.VMEM((_W,), jnp.float32),
            pltpu.VMEM((_W,), jnp.float32),
            pltpu.VMEM((_L,), jnp.float32),
            pltpu.VMEM((_L,), jnp.float32),
            pltpu.SemaphoreType.DMA,
            pltpu.SemaphoreType.DMA,
        ),
    )
    def sc_call(ts_hbm, pb_hbm, tb_hbm, mk_hbm,
                tst_hbm, pbt_hbm, tbt_hbm, mkt_hbm,
                loss_out, ts_out,
                xv0, xv1, pv0, pv1, tv0, tv1, mv0, mv1,
                l_v, t_v, sem0, sem1):
        wid = lax.axis_index("s") * NC + lax.axis_index("c")
        slots = ((xv0, pv0, tv0, mv0, sem0), (xv1, pv1, tv1, mv1, sem1))

        def unit_refs(u):
            u = jnp.minimum(u, n_su - 1)
            b = b0 + u // SU
            col = pl.multiple_of((u - (u // SU) * SU) * _W, 128)
            return (ts_hbm.at[b, :, pl.ds(col, _W)],
                    pb_hbm.at[b, :, pl.ds(col, _W)],
                    tb_hbm.at[b, :, pl.ds(col, _W)],
                    mk_hbm.at[b, pl.ds(col, _W)])

        def start(u, slot):
            xv, pv, tv, mv, sem = slots[slot]
            s_ts, s_pb, s_tb, s_mk = unit_refs(u)
            pltpu.async_copy(s_ts, xv, sem)
            pltpu.async_copy(s_pb, pv, sem)
            pltpu.async_copy(s_tb, tv, sem)
            pltpu.async_copy(s_mk, mv, sem)

        def wait(u, slot):
            xv, pv, tv, mv, sem = slots[slot]
            s_ts, s_pb, s_tb, s_mk = unit_refs(u)
            pltpu.make_async_copy(s_ts, xv, sem).wait()
            pltpu.make_async_copy(s_pb, pv, sem).wait()
            pltpu.make_async_copy(s_tb, tv, sem).wait()
            pltpu.make_async_copy(s_mk, mv, sem).wait()

        def block_sums(xv, base):
            UR = 8
            assert C % UR == 0

            def row_body(j, accs):
                r = j * UR
                out = []
                for g in range(8):
                    s = base + g * _L
                    x = [xv[r + i, pl.ds(s, _L)] for i in range(UR)]
                    t = ((x[0] + x[1]) + (x[2] + x[3])) + (
                        (x[4] + x[5]) + (x[6] + x[7]))
                    out.append(accs[g] + t)
                return tuple(out)

            zeros = tuple(jnp.zeros((_L,), jnp.float32) for _ in range(8))
            return lax.fori_loop(0, C // UR, row_body, zeros)

        def block_loss(slot_refs, base, ws, acc_l, acc_t):
            _, pv, tv, mv, _ = slot_refs
            for g in range(8):
                s = base + g * _L
                w = ws[g]
                px1 = pv[0, pl.ds(s, _L)]
                py1 = pv[1, pl.ds(s, _L)]
                px2 = pv[2, pl.ds(s, _L)]
                py2 = pv[3, pl.ds(s, _L)]
                tx1 = tv[0, pl.ds(s, _L)]
                ty1 = tv[1, pl.ds(s, _L)]
                tx2 = tv[2, pl.ds(s, _L)]
                ty2 = tv[3, pl.ds(s, _L)]
                iw = jnp.maximum(jnp.minimum(px2, tx2) - jnp.maximum(px1, tx1), 0.0)
                ih = jnp.maximum(jnp.minimum(py2, ty2) - jnp.maximum(py1, ty1), 0.0)
                inter = iw * ih
                area1 = jnp.maximum(px2 - px1, 0.0) * jnp.maximum(py2 - py1, 0.0)
                area2 = jnp.maximum(tx2 - tx1, 0.0) * jnp.maximum(ty2 - ty1, 0.0)
                union = area1 + area2 - inter + _EPS
                iou = inter / union
                cw = jnp.maximum(px2, tx2) - jnp.minimum(px1, tx1)
                ch = jnp.maximum(py2, ty2) - jnp.minimum(py1, ty1)
                c_area = cw * ch + _EPS
                giou = iou - (c_area - union) / c_area
                m = mv[pl.ds(s, _L)]
                acc_l = acc_l + (1.0 - giou) * w * m
                acc_t = acc_t + w
            return acc_l, acc_t

        def compute_slab(slot, valid, acc_l, acc_t):
            srefs = slots[slot]
            xv = srefs[0]

            def kk_body(kk, carry):
                acc_l, acc_t = carry
                base = kk * 128
                ws = block_sums(xv, base)
                ws = tuple(w * valid for w in ws)
                return block_loss(srefs, base, ws, acc_l, acc_t)

            return lax.fori_loop(0, _KB, kk_body, (acc_l, acc_t))

        def valid_of(u):
            return jnp.where(u < n_su, jnp.float32(1.0), jnp.float32(0.0))

        start(wid, 0)
        start(wid + NW, 1)

        def pair_body(j, carry):
            acc_l, acc_t = carry
            for slot in (0, 1):
                i = 2 * j + slot
                uu = wid + i * NW
                wait(uu, slot)

                @pl.when(i + 2 < trips)
                def _():
                    start(uu + 2 * NW, slot)

                acc_l, acc_t = compute_slab(slot, valid_of(uu), acc_l, acc_t)
            return acc_l, acc_t

        zero = jnp.zeros((_L,), jnp.float32)
        acc = lax.fori_loop(0, trips // 2, pair_body, (zero, zero))
        if trips % 2:
            acc_l, acc_t = acc
            wait(wid + (trips - 1) * NW, 0)
            acc = compute_slab(0, valid_of(wid + (trips - 1) * NW), acc_l, acc_t)

        def tail_body(i, carry):
            acc_l, acc_t = carry
            t = wid + i * NW
            b = jnp.minimum(t, nb - 1)
            xv, pv, tv, mv, _ = slots[0]
            pltpu.sync_copy(tst_hbm.at[b], xv.at[:, pl.ds(0, 128)])
            pltpu.sync_copy(pbt_hbm.at[b], pv.at[:, pl.ds(0, 128)])
            pltpu.sync_copy(tbt_hbm.at[b], tv.at[:, pl.ds(0, 128)])
            pltpu.sync_copy(mkt_hbm.at[b], mv.at[pl.ds(0, 128)])
            ws = block_sums(xv, 0)
            valid = jnp.where(t < nb, jnp.float32(1.0), jnp.float32(0.0))
            ws = tuple(w * valid for w in ws)
            return block_loss(slots[0], 0, ws, acc_l, acc_t)

        acc_l, acc_t = lax.fori_loop(0, ttrips, tail_body, acc)
        l_v[...] = acc_l
        t_v[...] = acc_t
        pltpu.sync_copy(l_v, loss_out.at[wid])
        pltpu.sync_copy(t_v, ts_out.at[wid])

    return sc_call


def _make_tc_call(B: int, N: int, C: int, btc: int):
    BB = 8

    def body(ts_ref, pb_ref, tb_ref, mk_ref, lo_ref, to_ref):
        b = pl.program_id(0)

        @pl.when(b == 0)
        def _():
            lo_ref[...] = jnp.zeros_like(lo_ref)
            to_ref[...] = jnp.zeros_like(to_ref)

        w = jnp.sum(ts_ref[...], axis=1)
        m = mk_ref[...]
        px1 = pb_ref[:, 0, :]
        py1 = pb_ref[:, 1, :]
        px2 = pb_ref[:, 2, :]
        py2 = pb_ref[:, 3, :]
        tx1 = tb_ref[:, 0, :]
        ty1 = tb_ref[:, 1, :]
        tx2 = tb_ref[:, 2, :]
        ty2 = tb_ref[:, 3, :]
        iw = jnp.maximum(jnp.minimum(px2, tx2) - jnp.maximum(px1, tx1), 0.0)
        ih = jnp.maximum(jnp.minimum(py2, ty2) - jnp.maximum(py1, ty1), 0.0)
        inter = iw * ih
        area1 = jnp.maximum(px2 - px1, 0.0) * jnp.maximum(py2 - py1, 0.0)
        area2 = jnp.maximum(tx2 - tx1, 0.0) * jnp.maximum(ty2 - ty1, 0.0)
        union = area1 + area2 - inter + _EPS
        iou = inter / union
        cw = jnp.maximum(px2, tx2) - jnp.minimum(px1, tx1)
        ch = jnp.maximum(py2, ty2) - jnp.minimum(py1, ty1)
        c_area = cw * ch + _EPS
        giou = iou - (c_area - union) / c_area
        lo_ref[...] += (1.0 - giou) * w * m
        to_ref[...] += w

    return pl.pallas_call(
        body,
        grid=(btc // BB,),
        compiler_params=pltpu.CompilerParams(
            vmem_limit_bytes=100 * 1024 * 1024),
        in_specs=[
            pl.BlockSpec((BB, C, N), lambda b: (b, 0, 0)),
            pl.BlockSpec((BB, 4, N), lambda b: (b, 0, 0)),
            pl.BlockSpec((BB, 4, N), lambda b: (b, 0, 0)),
            pl.BlockSpec((BB, N), lambda b: (b, 0)),
        ],
        out_specs=[
            pl.BlockSpec((BB, N), lambda b: (0, 0)),
            pl.BlockSpec((BB, N), lambda b: (0, 0)),
        ],
        out_shape=[
            jax.ShapeDtypeStruct((BB, N), jnp.float32),
            jax.ShapeDtypeStruct((BB, N), jnp.float32),
        ],
    )


_B_TC = 48


def kernel(pred_bboxes, target_bboxes, target_scores, mask_positive):
    B, N, C = target_scores.shape
    FULL = N // 128
    ncols = FULL * 128
    ts_t = target_scores.transpose(0, 2, 1)
    pb_t = pred_bboxes.transpose(0, 2, 1)
    tb_t = target_bboxes.transpose(0, 2, 1)
    mask_f = mask_positive.astype(jnp.float32)

    def pad_tail(x):
        tail = x[..., ncols:]
        return jnp.pad(tail, [(0, 0)] * (x.ndim - 1) + [(0, 128 - (N - ncols))])

    loss_p, ts_p = _make_sc_call(B, N, C, _B_TC)(
        ts_t, pb_t, tb_t, mask_f,
        pad_tail(ts_t[_B_TC:]), pad_tail(pb_t[_B_TC:]),
        pad_tail(tb_t[_B_TC:]), pad_tail(mask_f[_B_TC:]))
    lo_tc, to_tc = _make_tc_call(B, N, C, _B_TC)(
        ts_t, pb_t, tb_t, mask_f)
    loss = loss_p.sum() + lo_tc.sum()
    ts = ts_p.sum() + to_tc.sum()
    return jnp.where(ts > 1.0, loss / ts, loss)
